# baseline (speedup 1.0000x reference)
import jax
import jax.numpy as jnp
from jax.experimental import pallas as pl


_BT = 512


def _vq_block(z_ref, w_ref, q_ref, zq_ref):
    z = z_ref[...]
    w = w_ref[...]
    kk = w.shape[0]
    zw = jax.lax.dot_general(z, w, (((1,), (1,)), ((), ())),
                             preferred_element_type=jnp.float32)
    zsq = jnp.sum(z * z, axis=1, keepdims=True)
    wsq = jnp.sum(w * w, axis=1)[None, :]
    dist = zsq - 2.0 * zw + wsq
    minv = jnp.min(dist, axis=1, keepdims=True)
    iota = jax.lax.broadcasted_iota(jnp.int32, (z.shape[0], kk), 1)
    q = jnp.min(jnp.where(dist == minv, iota, kk), axis=1).astype(jnp.int32)
    q_ref[0, 0, :] = q
    onehot = (iota == q[:, None]).astype(jnp.float32)
    zq_ref[...] = jax.lax.dot_general(onehot, w, (((1,), (0,)), ((), ())),
                                      preferred_element_type=jnp.float32,
                                      precision=jax.lax.Precision.HIGHEST)


def kernel(z_e, weights):
    N, D, H, W = z_e.shape
    K = weights.shape[0]
    z = jnp.transpose(z_e, (0, 2, 3, 1)).reshape(-1, D)
    T = z.shape[0]
    nb = T // _BT
    q3, zq = pl.pallas_call(
        _vq_block,
        grid=(nb,),
        in_specs=[
            pl.BlockSpec((_BT, D), lambda i: (i, 0)),
            pl.BlockSpec((K, D), lambda i: (0, 0)),
        ],
        out_specs=[
            pl.BlockSpec((1, 1, _BT), lambda i: (i, 0, 0)),
            pl.BlockSpec((_BT, D), lambda i: (i, 0)),
        ],
        out_shape=[
            jax.ShapeDtypeStruct((nb, 1, _BT), jnp.int32),
            jax.ShapeDtypeStruct((T, D), jnp.float32),
        ],
    )(z, weights)
    q = q3.reshape(N, H, W)
    z_q = zq.reshape(N, H, W, D).transpose(0, 3, 1, 2)
    return q, z_q


# speedup vs baseline: 1.2246x; 1.2246x over previous
import jax
import jax.numpy as jnp
from jax.experimental import pallas as pl


def _vq_block(z_ref, w_ref, q_ref, zq_ref):
    zt = z_ref[0]
    w = w_ref[...]
    kk = w.shape[0]
    z = zt.T
    zw = jax.lax.dot_general(z, w, (((1,), (1,)), ((), ())),
                             preferred_element_type=jnp.float32)
    zsq = jnp.sum(z * z, axis=1, keepdims=True)
    wsq = jnp.sum(w * w, axis=1)[None, :]
    dist = zsq - 2.0 * zw + wsq
    minv = jnp.min(dist, axis=1, keepdims=True)
    iota = jax.lax.broadcasted_iota(jnp.int32, dist.shape, 1)
    q = jnp.min(jnp.where(dist == minv, iota, kk), axis=1).astype(jnp.int32)
    q_ref[0, 0, :] = q
    onehot = (iota == q[:, None]).astype(jnp.float32).astype(jnp.bfloat16)
    w_hi = w.astype(jnp.bfloat16)
    r1 = w - w_hi.astype(jnp.float32)
    w_mid = r1.astype(jnp.bfloat16)
    w_lo = (r1 - w_mid.astype(jnp.float32)).astype(jnp.bfloat16)
    dims = (((1,), (0,)), ((), ()))
    zq = (jax.lax.dot_general(onehot, w_hi, dims,
                              preferred_element_type=jnp.float32)
          + jax.lax.dot_general(onehot, w_mid, dims,
                                preferred_element_type=jnp.float32)) \
        + jax.lax.dot_general(onehot, w_lo, dims,
                              preferred_element_type=jnp.float32)
    zq_ref[0] = zq.T


def kernel(z_e, weights):
    N, D, H, W = z_e.shape
    K = weights.shape[0]
    BT = H * W
    z3 = z_e.reshape(N, D, BT)
    q3, zq = pl.pallas_call(
        _vq_block,
        grid=(N,),
        in_specs=[
            pl.BlockSpec((1, D, BT), lambda i: (i, 0, 0)),
            pl.BlockSpec((K, D), lambda i: (0, 0)),
        ],
        out_specs=[
            pl.BlockSpec((1, 1, BT), lambda i: (i, 0, 0)),
            pl.BlockSpec((1, D, BT), lambda i: (i, 0, 0)),
        ],
        out_shape=[
            jax.ShapeDtypeStruct((N, 1, BT), jnp.int32),
            jax.ShapeDtypeStruct((N, D, BT), jnp.float32),
        ],
    )(z3, weights)
    q = q3.reshape(N, H, W)
    z_q = zq.reshape(N, D, H, W)
    return q, z_q


# speedup vs baseline: 1.8483x; 1.5092x over previous
import jax
import jax.numpy as jnp
from jax.experimental import pallas as pl


def _vq_block(z_ref, w_ref, zsq_ref, q_ref, zq_ref):
    zt = z_ref[0]
    w = w_ref[...]
    kk = w.shape[0]
    zsq = zsq_ref[0]
    zwt = jax.lax.dot_general(w, zt, (((1,), (0,)), ((), ())),
                              preferred_element_type=jnp.float32)
    wsq = jnp.sum(w * w, axis=1)[:, None]
    dist = zsq - 2.0 * zwt + wsq
    minv = jnp.min(dist, axis=0, keepdims=True)
    iota = jax.lax.broadcasted_iota(jnp.int32, dist.shape, 0)
    q = jnp.min(jnp.where(dist == minv, iota, kk), axis=0).astype(jnp.int32)
    q_ref[0, 0, :] = q
    onehot = (iota == q[None, :]).astype(jnp.float32)
    w_hi = w.astype(jnp.bfloat16).astype(jnp.float32)
    r1 = w - w_hi
    w_mid = r1.astype(jnp.bfloat16).astype(jnp.float32)
    w_lo = r1 - w_mid
    w3 = jnp.concatenate([w_hi, w_mid, w_lo], axis=1)
    zq3 = jax.lax.dot_general(w3, onehot, (((0,), (0,)), ((), ())),
                              preferred_element_type=jnp.float32)
    dd = w.shape[1]
    zq_ref[0] = (zq3[:dd] + zq3[dd:2 * dd]) + zq3[2 * dd:]


def kernel(z_e, weights):
    N, D, H, W = z_e.shape
    K = weights.shape[0]
    BT = H * W
    z3 = z_e.reshape(N, D, BT)
    zsq3 = (z_e ** 2).sum(axis=1).reshape(N, 1, BT)
    q3, zq = pl.pallas_call(
        _vq_block,
        grid=(N,),
        in_specs=[
            pl.BlockSpec((1, D, BT), lambda i: (i, 0, 0)),
            pl.BlockSpec((K, D), lambda i: (0, 0)),
            pl.BlockSpec((1, 1, BT), lambda i: (i, 0, 0)),
        ],
        out_specs=[
            pl.BlockSpec((1, 1, BT), lambda i: (i, 0, 0)),
            pl.BlockSpec((1, D, BT), lambda i: (i, 0, 0)),
        ],
        out_shape=[
            jax.ShapeDtypeStruct((N, 1, BT), jnp.int32),
            jax.ShapeDtypeStruct((N, D, BT), jnp.float32),
        ],
    )(z3, weights, zsq3)
    q = q3.reshape(N, H, W)
    z_q = zq.reshape(N, D, H, W)
    return q, z_q


# speedup vs baseline: 1.9901x; 1.0767x over previous
import jax
import jax.numpy as jnp
from jax.experimental import pallas as pl
from jax.experimental.pallas import tpu as pltpu


def _vq_block(z_ref, w_ref, zsq_ref, q_ref, zq_ref,
              w2_ref, w3_ref, wsq_ref, iotaf_ref):
    kk = w_ref.shape[0]

    @pl.when(pl.program_id(0) == 0)
    def _init():
        w = w_ref[...]
        w2_ref[...] = w + w
        w_hi = w.astype(jnp.bfloat16).astype(jnp.float32)
        r1 = w - w_hi
        w_mid = r1.astype(jnp.bfloat16).astype(jnp.float32)
        w_lo = r1 - w_mid
        w3_ref[...] = jnp.concatenate([w_hi, w_mid, w_lo], axis=1)
        wsq_ref[...] = jnp.sum(w * w, axis=1)[:, None]
        iotaf_ref[...] = jax.lax.broadcasted_iota(
            jnp.int32, iotaf_ref.shape, 0).astype(jnp.float32)

    zt = z_ref[0]
    zsq = zsq_ref[0]
    zw2 = jax.lax.dot_general(w2_ref[...], zt, (((1,), (0,)), ((), ())),
                              preferred_element_type=jnp.float32)
    dist = zsq - zw2 + wsq_ref[...]
    minv = jnp.min(dist, axis=0, keepdims=True)
    iotaf = iotaf_ref[...]
    qf = jnp.min(jnp.where(dist == minv, iotaf, float(kk)), axis=0)
    q_ref[0, 0, :] = qf.astype(jnp.int32)
    onehot = (iotaf == qf[None, :]).astype(jnp.float32)
    zq3 = jax.lax.dot_general(w3_ref[...], onehot, (((0,), (0,)), ((), ())),
                              preferred_element_type=jnp.float32)
    dd = zt.shape[0]
    zq_ref[0] = (zq3[:dd] + zq3[dd:2 * dd]) + zq3[2 * dd:]


def kernel(z_e, weights):
    N, D, H, W = z_e.shape
    K = weights.shape[0]
    BT = H * W
    z3 = z_e.reshape(N, D, BT)
    zsq3 = (z_e ** 2).sum(axis=1).reshape(N, 1, BT)
    q3, zq = pl.pallas_call(
        _vq_block,
        grid=(N,),
        in_specs=[
            pl.BlockSpec((1, D, BT), lambda i: (i, 0, 0)),
            pl.BlockSpec((K, D), lambda i: (0, 0)),
            pl.BlockSpec((1, 1, BT), lambda i: (i, 0, 0)),
        ],
        out_specs=[
            pl.BlockSpec((1, 1, BT), lambda i: (i, 0, 0)),
            pl.BlockSpec((1, D, BT), lambda i: (i, 0, 0)),
        ],
        out_shape=[
            jax.ShapeDtypeStruct((N, 1, BT), jnp.int32),
            jax.ShapeDtypeStruct((N, D, BT), jnp.float32),
        ],
        scratch_shapes=[
            pltpu.VMEM((K, D), jnp.float32),
            pltpu.VMEM((K, 3 * D), jnp.float32),
            pltpu.VMEM((K, 1), jnp.float32),
            pltpu.VMEM((K, BT), jnp.float32),
        ],
    )(z3, weights, zsq3)
    q = q3.reshape(N, H, W)
    z_q = zq.reshape(N, D, H, W)
    return q, z_q
